# Initial kernel scaffold; baseline (speedup 1.0000x reference)
import jax
import jax.numpy as jnp
from jax.experimental import pallas as pl


def kernel(x_s, x_t, edge_index, edge_attr, u, batch_t, W1a, b1a, W1b, b1b, W2a, b2a, W2b, b2b):
    raise NotImplementedError("write your pallas kernel here")


# speedup vs baseline: 2.9425x; 2.9425x over previous
import functools

import jax
import jax.numpy as jnp
from jax import lax
from jax.experimental import pallas as pl
from jax.experimental.pallas import tpu as pltpu
from jax.experimental.pallas import tpu_sc as plsc

D = 32
CHUNK = 128
NC = 2
NS = 16
NW = NC * NS


def _table_body(x_ref, w_ref, b_ref, o_ref):
    o_ref[...] = (
        jnp.dot(x_ref[...], w_ref[...], preferred_element_type=jnp.float32, precision=lax.Precision.HIGHEST)
        + b_ref[...]
    )


def _make_table(x, w, b, rows_out, block_rows):
    n = x.shape[0]
    grid = n // block_rows
    return pl.pallas_call(
        _table_body,
        grid=(grid,),
        in_specs=[
            pl.BlockSpec((block_rows, x.shape[1]), lambda i: (i, 0)),
            pl.BlockSpec((x.shape[1], D), lambda i: (0, 0)),
            pl.BlockSpec((1, D), lambda i: (0, 0)),
        ],
        out_specs=pl.BlockSpec((block_rows, D), lambda i: (i, 0)),
        out_shape=jax.ShapeDtypeStruct((rows_out, D), jnp.float32),
    )(x, w, b)


def _make_sc_edge_pass(e_pad, acc_rows):
    per_tile = e_pad // NW
    iters = per_tile // CHUNK
    tile_rows = acc_rows // NS
    mesh = plsc.VectorSubcoreMesh(core_axis_name="c", subcore_axis_name="s")

    @functools.partial(
        pl.kernel,
        mesh=mesh,
        compiler_params=pltpu.CompilerParams(use_tc_tiling_on_sc=False),
        out_type=jax.ShapeDtypeStruct((NC, acc_rows, D), jnp.float32),
        scratch_types=[
            pltpu.VMEM((CHUNK,), jnp.int32),
            pltpu.VMEM((CHUNK,), jnp.int32),
            pltpu.VMEM((CHUNK, D), jnp.float32),
            pltpu.VMEM((CHUNK, D), jnp.float32),
            pltpu.VMEM_SHARED((acc_rows, D), jnp.float32),
            pltpu.SemaphoreType.DMA,
        ],
    )
    def sc_edge_pass(src_hbm, tgt_hbm, p_hbm, q_hbm, zeros_hbm, out_hbm,
                     src_v, tgt_v, rows_v, q_v, acc, sem):
        c = lax.axis_index("c")
        s = lax.axis_index("s")
        w = c * NS + s
        pltpu.sync_copy(zeros_hbm, acc.at[pl.ds(s * tile_rows, tile_rows)])
        plsc.subcore_barrier()

        def step(j, carry):
            base = w * per_tile + j * CHUNK
            pltpu.sync_copy(src_hbm.at[pl.ds(base, CHUNK)], src_v)
            pltpu.sync_copy(tgt_hbm.at[pl.ds(base, CHUNK)], tgt_v)
            pltpu.async_copy(p_hbm.at[src_v], rows_v, sem).wait()
            pltpu.sync_copy(q_hbm.at[pl.ds(base, CHUNK)], q_v)

            def row(r, carry2):
                for cc in range(D // 16):
                    sl = pl.ds(cc * 16, 16)
                    v = rows_v[r, sl] + q_v[r, sl]
                    rows_v[r, sl] = jnp.maximum(v, 0.1 * v)
                return carry2

            lax.fori_loop(0, CHUNK, row, 0)
            pltpu.sync_copy(rows_v, acc.at[tgt_v], add=True)
            return carry

        lax.fori_loop(0, iters, step, 0)
        plsc.subcore_barrier()
        pltpu.sync_copy(acc.at[pl.ds(s * tile_rows, tile_rows)],
                        out_hbm.at[c, pl.ds(s * tile_rows, tile_rows)])

    return sc_edge_pass


def _final_body(part_ref, xt_ref, batch_ref, u_ref, w1b_ref, b1b_ref,
                w2a_ref, b2a_ref, w2b_ref, b2b_ref, o_ref):
    S = part_ref[0] + part_ref[1]
    s20 = S[:, :20]
    cnt = S[:, 20:21]
    w2a = w2a_ref[...]
    w2a_x = w2a[:, :5]
    w2a_a = w2a[:, 5:25]
    w2a_u = w2a[:, 25:]
    wc = jnp.dot(w2a_a, w1b_ref[...], preferred_element_type=jnp.float32, precision=lax.Precision.HIGHEST)
    bc = jnp.dot(b1b_ref[...], w2a_a.T, preferred_element_type=jnp.float32, precision=lax.Precision.HIGHEST)
    cu = jnp.dot(u_ref[...], w2a_u.T, preferred_element_type=jnp.float32, precision=lax.Precision.HIGHEST)
    batch = batch_ref[0, 0, :]
    oh = (batch[:, None]
          == lax.broadcasted_iota(jnp.int32, (batch.shape[0], 64), 1)
          ).astype(jnp.float32)
    y = (jnp.dot(xt_ref[...], w2a_x.T, preferred_element_type=jnp.float32, precision=lax.Precision.HIGHEST)
         + jnp.dot(s20, wc.T, preferred_element_type=jnp.float32, precision=lax.Precision.HIGHEST)
         + cnt * bc
         + jnp.dot(oh, cu, preferred_element_type=jnp.float32, precision=lax.Precision.HIGHEST)
         + b2a_ref[...])
    y = jnp.maximum(y, 0.1 * y)
    o_ref[...] = (jnp.dot(y, w2b_ref[...].T, preferred_element_type=jnp.float32, precision=lax.Precision.HIGHEST)
                  + b2b_ref[...])


def _node_update(partials, xt_pad, batch3, u, w1b, b1b2, w2a, b2a2, w2b, b2b2):
    acc_rows = partials.shape[1]
    grid = 8
    r = acc_rows // grid
    full = lambda i: (0, 0)
    return pl.pallas_call(
        _final_body,
        grid=(grid,),
        in_specs=[
            pl.BlockSpec((NC, r, D), lambda i: (0, i, 0)),
            pl.BlockSpec((r, 5), lambda i: (i, 0)),
            pl.BlockSpec((1, 1, r), lambda i: (i, 0, 0)),
            pl.BlockSpec((64, 10), full),
            pl.BlockSpec((20, 20), full),
            pl.BlockSpec((1, 20), full),
            pl.BlockSpec((5, 35), full),
            pl.BlockSpec((1, 5), full),
            pl.BlockSpec((5, 5), full),
            pl.BlockSpec((1, 5), full),
        ],
        out_specs=pl.BlockSpec((r, 5), lambda i: (i, 0)),
        out_shape=jax.ShapeDtypeStruct((acc_rows, 5), jnp.float32),
    )(partials, xt_pad, batch3, u, w1b, b1b2, w2a, b2a2, w2b, b2b2)


def kernel(x_s, x_t, edge_index, edge_attr, u, batch_t,
           W1a, b1a, W1b, b1b, W2a, b2a, W2b, b2b):
    n_s, f_xs = x_s.shape
    n_t = x_t.shape[0]
    e = edge_attr.shape[0]

    src = edge_index[0].astype(jnp.int32)
    tgt = edge_index[1].astype(jnp.int32)
    batch32 = batch_t.astype(jnp.int32)

    w_src = jnp.zeros((f_xs, D), jnp.float32).at[:, :20].set(W1a[:, :f_xs].T)
    b_src = jnp.zeros((1, D), jnp.float32).at[0, :20].set(b1a).at[0, 20].set(1.0)
    w_edge = jnp.zeros((f_xs, D), jnp.float32).at[:, :20].set(W1a[:, f_xs:].T)

    p_tab = _make_table(x_s, w_src, b_src, n_s, 2000)

    e_pad = -(-e // (NW * CHUNK)) * (NW * CHUNK)
    q_tab = _make_table(edge_attr, w_edge, jnp.zeros((1, D), jnp.float32),
                        e_pad, 8000)

    dummy = n_t
    acc_rows = -(-(n_t + 1) // 1024) * 1024
    src_pad = jnp.concatenate([src, jnp.zeros((e_pad - e,), jnp.int32)])
    tgt_pad = jnp.concatenate([tgt, jnp.full((e_pad - e,), dummy, jnp.int32)])
    zeros_blk = jnp.zeros((acc_rows // NS, D), jnp.float32)

    sc_pass = _make_sc_edge_pass(e_pad, acc_rows)
    partials = sc_pass(src_pad, tgt_pad, p_tab, q_tab, zeros_blk)

    xt_pad = jnp.pad(x_t, ((0, acc_rows - n_t), (0, 0)))
    batch3 = jnp.pad(batch32, (0, acc_rows - n_t)).reshape(8, 1, acc_rows // 8)

    out_pad = _node_update(partials, xt_pad, batch3, u, W1b,
                           b1b.reshape(1, 20), W2a, b2a.reshape(1, 5),
                           W2b, b2b.reshape(1, 5))
    return out_pad[:n_t]


# speedup vs baseline: 3.9144x; 1.3303x over previous
import functools

import jax
import jax.numpy as jnp
from jax import lax
from jax.experimental import pallas as pl
from jax.experimental.pallas import tpu as pltpu
from jax.experimental.pallas import tpu_sc as plsc

D = 24
D_OFFS = (0, 8)
CHUNK = 80
NC = 2
NS = 16
NW = NC * NS


def _r(x):
    return x.astype(jnp.bfloat16).astype(jnp.float32)


def _table_body(x_ref, w_ref, b_ref, o_ref):
    o_ref[...] = (
        jnp.dot(_r(x_ref[...]), _r(w_ref[...]),
                preferred_element_type=jnp.float32,
                precision=lax.Precision.HIGHEST)
        + b_ref[...]
    )


def _make_table(x, w, b, rows_out, block_rows):
    n = x.shape[0]
    grid = n // block_rows
    return pl.pallas_call(
        _table_body,
        grid=(grid,),
        in_specs=[
            pl.BlockSpec((block_rows, x.shape[1]), lambda i: (i, 0)),
            pl.BlockSpec((x.shape[1], D), lambda i: (0, 0)),
            pl.BlockSpec((1, D), lambda i: (0, 0)),
        ],
        out_specs=pl.BlockSpec((block_rows, D), lambda i: (i, 0)),
        out_shape=jax.ShapeDtypeStruct((rows_out, D), jnp.float32),
    )(x, w, b)


RING = 5


def _make_sc_edge_pass(e_pad, acc_rows):
    per_tile = e_pad // NW
    t_chunks = per_tile // CHUNK
    groups = t_chunks // RING
    tile_rows = acc_rows // NS
    mesh = plsc.VectorSubcoreMesh(core_axis_name="c", subcore_axis_name="s")

    scratch = []
    for _ in range(RING):
        scratch += [
            pltpu.VMEM((CHUNK,), jnp.int32),
            pltpu.VMEM((CHUNK,), jnp.int32),
            pltpu.VMEM((CHUNK,), jnp.int32),
            pltpu.VMEM((CHUNK, D), jnp.float32),
            pltpu.VMEM((CHUNK, D), jnp.float32),
            pltpu.VMEM((CHUNK, D), jnp.float32),
        ]
    scratch.append(pltpu.VMEM_SHARED((acc_rows, D), jnp.float32))
    scratch += [pltpu.SemaphoreType.DMA] * (3 * RING)

    @functools.partial(
        pl.kernel,
        mesh=mesh,
        compiler_params=pltpu.CompilerParams(use_tc_tiling_on_sc=False),
        out_type=jax.ShapeDtypeStruct((NC, acc_rows, D), jnp.float32),
        scratch_types=scratch,
    )
    def sc_edge_pass(src_hbm, tgt_hbm, p_hbm, q_hbm, zeros_hbm, out_hbm, *scr):
        src = [scr[6 * k + 0] for k in range(RING)]
        tgt = [scr[6 * k + 1] for k in range(RING)]
        stgt = [scr[6 * k + 2] for k in range(RING)]
        rows = [scr[6 * k + 3] for k in range(RING)]
        q = [scr[6 * k + 4] for k in range(RING)]
        h = [scr[6 * k + 5] for k in range(RING)]
        acc = scr[6 * RING]
        isem = [scr[6 * RING + 1 + k] for k in range(RING)]
        gqsem = [scr[6 * RING + 1 + RING + k] for k in range(RING)]
        ssem = [scr[6 * RING + 1 + 2 * RING + k] for k in range(RING)]

        c = lax.axis_index("c")
        s = lax.axis_index("s")
        w = c * NS + s
        tile_base = w * per_tile

        def issue_idx(j, b):
            base = tile_base + j * CHUNK
            pltpu.async_copy(src_hbm.at[pl.ds(base, CHUNK)], src[b], isem[b])
            pltpu.async_copy(tgt_hbm.at[pl.ds(base, CHUNK)], tgt[b], isem[b])

        def wait_idx(b):
            pltpu.make_async_copy(src_hbm.at[pl.ds(0, CHUNK)], src[b],
                                  isem[b]).wait()
            pltpu.make_async_copy(tgt_hbm.at[pl.ds(0, CHUNK)], tgt[b],
                                  isem[b]).wait()

        def issue_gq(j, b):
            base = tile_base + j * CHUNK
            pltpu.async_copy(p_hbm.at[src[b]], rows[b], gqsem[b])
            pltpu.async_copy(q_hbm.at[pl.ds(base, CHUNK)], q[b], gqsem[b])

        def wait_gq(b):
            pltpu.make_async_copy(p_hbm.at[src[b]], rows[b], gqsem[b]).wait()
            pltpu.make_async_copy(q_hbm.at[pl.ds(0, CHUNK)], q[b],
                                  gqsem[b]).wait()

        def wait_scatter(b):
            pltpu.make_async_copy(h[b], acc.at[stgt[b]], ssem[b]).wait()

        def compute(b):
            def rowfn(r, carry):
                for rr in range(4):
                    for off in D_OFFS:
                        sl = pl.ds(off, 16)
                        v = rows[b][4 * r + rr, sl] + q[b][4 * r + rr, sl]
                        hh = jnp.maximum(v, 0.1 * v)
                        u = lax.bitcast_convert_type(hh, jnp.int32)
                        tie = lax.shift_right_logical(u, 16) & 1
                        u = (u + 32767 + tie) & jnp.int32(-65536)
                        h[b][4 * r + rr, sl] = lax.bitcast_convert_type(
                            u, jnp.float32)
                return carry
            lax.fori_loop(0, CHUNK // 4, rowfn, 0)

        def body(j, b, *, first, last):
            wait_gq(b)
            if not first:
                wait_scatter(b)
            for k in range(CHUNK // 16):
                sl = pl.ds(16 * k, 16)
                stgt[b][sl] = tgt[b][sl]
            if not last:
                issue_idx(j + RING, b)
            compute(b)
            pltpu.async_copy(h[b], acc.at[stgt[b]], ssem[b], add=True)
            if not last:
                b2 = (b + 2) % RING
                wait_idx(b2)
                issue_gq(j + 2, b2)
            else:
                if b <= RING - 3:
                    b2 = (b + 2) % RING
                    wait_idx(b2)
                    issue_gq(j + 2, b2)

        pltpu.sync_copy(zeros_hbm, acc.at[pl.ds(s * tile_rows, tile_rows)])
        plsc.subcore_barrier()

        for b in range(RING):
            issue_idx(b, b)
        for b in range(2):
            wait_idx(b)
            issue_gq(b, b)
        for b in range(RING):
            body(b, b, first=True, last=False)

        def group(g, carry):
            for b in range(RING):
                body(g * RING + b, b, first=False, last=False)
            return carry

        lax.fori_loop(1, groups - 1, group, 0)

        for b in range(RING):
            body((groups - 1) * RING + b, b, first=False, last=True)
        for b in range(RING):
            wait_scatter(b)

        plsc.subcore_barrier()
        pltpu.sync_copy(acc.at[pl.ds(s * tile_rows, tile_rows)],
                        out_hbm.at[c, pl.ds(s * tile_rows, tile_rows)])

    return sc_edge_pass


def _final_body(part_ref, xt_ref, batch_ref, u_ref, w1b_ref, b1b_ref,
                w2a_ref, b2a_ref, w2b_ref, b2b_ref, o_ref):
    hp = dict(preferred_element_type=jnp.float32,
              precision=lax.Precision.HIGHEST)
    S = part_ref[0] + part_ref[1]
    s20 = S[:, :20]
    cnt = S[:, 20:21]
    w2a = w2a_ref[...]
    w2a_x = w2a[:, :5]
    w2a_a = w2a[:, 5:25]
    w2a_u = w2a[:, 25:]
    a = jnp.dot(s20, _r(w1b_ref[...]).T, **hp) + cnt * b1b_ref[...]
    cu = jnp.dot(_r(u_ref[...]), _r(w2a_u).T, **hp)
    batch = batch_ref[0, 0, :]
    oh = (batch[:, None]
          == lax.broadcasted_iota(jnp.int32, (batch.shape[0], 64), 1)
          ).astype(jnp.float32)
    y = (jnp.dot(_r(xt_ref[...]), _r(w2a_x).T, **hp)
         + jnp.dot(_r(a), _r(w2a_a).T, **hp)
         + jnp.dot(oh, cu, **hp)
         + b2a_ref[...])
    y = jnp.maximum(y, 0.1 * y)
    o_ref[...] = jnp.dot(_r(y), _r(w2b_ref[...]).T, **hp) + b2b_ref[...]


def _node_update(partials, xt_pad, batch3, u, w1b, b1b2, w2a, b2a2, w2b, b2b2):
    acc_rows = partials.shape[1]
    grid = 8
    r = acc_rows // grid
    full = lambda i: (0, 0)
    return pl.pallas_call(
        _final_body,
        grid=(grid,),
        in_specs=[
            pl.BlockSpec((NC, r, D), lambda i: (0, i, 0)),
            pl.BlockSpec((r, 5), lambda i: (i, 0)),
            pl.BlockSpec((1, 1, r), lambda i: (i, 0, 0)),
            pl.BlockSpec((64, 10), full),
            pl.BlockSpec((20, 20), full),
            pl.BlockSpec((1, 20), full),
            pl.BlockSpec((5, 35), full),
            pl.BlockSpec((1, 5), full),
            pl.BlockSpec((5, 5), full),
            pl.BlockSpec((1, 5), full),
        ],
        out_specs=pl.BlockSpec((r, 5), lambda i: (i, 0)),
        out_shape=jax.ShapeDtypeStruct((acc_rows, 5), jnp.float32),
    )(partials, xt_pad, batch3, u, w1b, b1b2, w2a, b2a2, w2b, b2b2)


def kernel(x_s, x_t, edge_index, edge_attr, u, batch_t,
           W1a, b1a, W1b, b1b, W2a, b2a, W2b, b2b):
    n_s, f_xs = x_s.shape
    n_t = x_t.shape[0]
    e = edge_attr.shape[0]

    src = edge_index[0].astype(jnp.int32)
    tgt = edge_index[1].astype(jnp.int32)
    batch32 = batch_t.astype(jnp.int32)

    w_src = jnp.zeros((f_xs, D), jnp.float32).at[:, :20].set(W1a[:, :f_xs].T)
    b_src = jnp.zeros((1, D), jnp.float32).at[0, :20].set(b1a).at[0, 20].set(1.0)
    w_edge = jnp.zeros((f_xs, D), jnp.float32).at[:, :20].set(W1a[:, f_xs:].T)

    p_tab = _make_table(x_s, w_src, b_src, n_s, 2000)

    e_pad = -(-e // (NW * CHUNK * RING)) * (NW * CHUNK * RING)
    q_tab = _make_table(edge_attr, w_edge, jnp.zeros((1, D), jnp.float32),
                        e_pad, 8000)

    acc_rows = -(-(n_t + 1) // 1024) * 1024
    if e_pad != e:
        src = jnp.concatenate([src, jnp.zeros((e_pad - e,), jnp.int32)])
        tgt = jnp.concatenate([tgt, jnp.full((e_pad - e,), n_t, jnp.int32)])
    zeros_blk = jnp.zeros((acc_rows // NS, D), jnp.float32)

    sc_pass = _make_sc_edge_pass(e_pad, acc_rows)
    partials = sc_pass(src, tgt, p_tab, q_tab, zeros_blk)

    xt_pad = jnp.pad(x_t, ((0, acc_rows - n_t), (0, 0)))
    batch3 = jnp.pad(batch32, (0, acc_rows - n_t)).reshape(8, 1, acc_rows // 8)

    out_pad = _node_update(partials, xt_pad, batch3, u, W1b,
                           b1b.reshape(1, 20), W2a, b2a.reshape(1, 5),
                           W2b, b2b.reshape(1, 5))
    return out_pad[:n_t]
